# baseline (speedup 1.0000x reference)
import jax
import jax.numpy as jnp
from jax import lax
from jax.experimental import pallas as pl
from jax.experimental.pallas import tpu as pltpu
from jax.experimental.pallas import tpu_sc as plsc

N = 10000
E = 320000
D = 128
H = 64
G = 64
C = 10

NS = 16
L = 16

NPAD = 10240
NSL = NPAD // NS
CNT_OFF = G * N
ASZ = G * N + 128
SL = ASZ // NS
SL2 = 20008
EC = E // NS
CB = 1024
RB = CB // 128
NB = (EC + CB - 1) // CB
ZB = 4016


def _invsqrt(v):
    i = lax.bitcast_convert_type(v, jnp.int32)
    i = jnp.int32(0x5F3759DF) - (i >> 1)
    y = lax.bitcast_convert_type(i, jnp.float32)
    for _ in range(3):
        y = y * (1.5 - 0.5 * v * y * y)
    return y


def _zero_rows(buf, zero16):
    for r in range(RB):
        for q in range(128 // L):
            buf[r, pl.ds(q * L, L)] = zero16


def _fire(val_b, idx_b, dst_sp, sem):
    for r in range(RB):
        pltpu.async_copy(val_b.at[r], dst_sp.at[idx_b.at[r]], sem, add=True)


def _drain(val_b, idx_b, dst_sp, sem):
    for r in range(RB):
        pltpu.make_async_copy(val_b.at[r], dst_sp.at[idx_b.at[r]], sem).wait()


def _sc_body(src_hbm, dst_hbm, batch_hbm, out_hbm,
             batch_v, dinv_v, edge_v, degsl_v, idx0, val0, idx1, val1,
             zeros_v, stage_v, sem0, sem1, a_sp, deg_sp, dinv_sp):
    s = lax.axis_index("s")
    iota = lax.iota(jnp.int32, L)
    zero16 = jnp.zeros((L,), jnp.float32)
    izero16 = jnp.zeros((L,), jnp.int32)

    pltpu.sync_copy(batch_hbm, batch_v)
    pltpu.sync_copy(src_hbm.at[pl.ds(s * EC, EC)], edge_v.at[pl.ds(0, EC)])
    pltpu.sync_copy(dst_hbm.at[pl.ds(s * EC, EC)], edge_v.at[pl.ds(EC, EC)])

    def zloop(i, _):
        zeros_v[pl.ds(i * L, L)] = zero16
        return 0
    lax.fori_loop(0, ZB // L, zloop, 0)

    base = s * SL
    for j in range(9):
        pltpu.sync_copy(zeros_v.at[pl.ds(0, 4000)],
                        a_sp.at[pl.ds(base + j * 4000, 4000)])
    pltpu.sync_copy(zeros_v.at[pl.ds(0, SL - 36000)],
                    a_sp.at[pl.ds(base + 36000, SL - 36000)])
    pltpu.sync_copy(zeros_v.at[pl.ds(0, NSL)], deg_sp.at[pl.ds(s * NSL, NSL)])
    for idx_b, val_b in ((idx0, val0), (idx1, val1)):
        _zero_rows(val_b, zero16)
        for r in range(RB):
            for q in range(128 // L):
                idx_b[r, pl.ds(q * L, L)] = izero16
    plsc.subcore_barrier()

    def fill(idx_b, val_b, b, with_norm):
        for k in range(CB // L):
            e0 = b * CB + k * L
            e0c = jnp.minimum(e0, EC - L)
            ok = (e0 + iota) < EC
            r, col = k // 8, (k % 8) * L
            d16 = edge_v[pl.ds(EC + e0c, L)]
            if with_norm:
                s16 = edge_v[pl.ds(e0c, L)]
                dvs = plsc.load_gather(dinv_v, [s16])
                dvd = plsc.load_gather(dinv_v, [d16])
                g16 = plsc.load_gather(batch_v, [d16])
                idx_b[r, pl.ds(col, L)] = g16 * N + s16
                val_b[r, pl.ds(col, L)] = jnp.where(ok, dvs * dvd, 0.0)
            else:
                idx_b[r, pl.ds(col, L)] = d16
                val_b[r, pl.ds(col, L)] = jnp.where(ok, 1.0, 0.0)

    _fire(val0, idx0, deg_sp, sem0)
    _fire(val1, idx1, deg_sp, sem1)

    def p1(i, _):
        _drain(val0, idx0, deg_sp, sem0)
        fill(idx0, val0, 2 * i, False)
        _fire(val0, idx0, deg_sp, sem0)
        _drain(val1, idx1, deg_sp, sem1)
        fill(idx1, val1, 2 * i + 1, False)
        _fire(val1, idx1, deg_sp, sem1)
        return 0
    lax.fori_loop(0, NB // 2, p1, 0)
    _drain(val0, idx0, deg_sp, sem0)
    _drain(val1, idx1, deg_sp, sem1)
    plsc.subcore_barrier()

    pltpu.sync_copy(deg_sp.at[pl.ds(s * NSL, NSL)], degsl_v)

    def dloop(jj, _):
        dg = degsl_v[pl.ds(jj * L, L)] + 1.0
        dinv_v[pl.ds(s * NSL + jj * L, L)] = _invsqrt(dg)
        return 0
    lax.fori_loop(0, NSL // L, dloop, 0)
    pltpu.sync_copy(dinv_v.at[pl.ds(s * NSL, NSL)],
                    dinv_sp.at[pl.ds(s * NSL, NSL)])
    plsc.subcore_barrier()
    pltpu.sync_copy(dinv_sp, dinv_v)

    _zero_rows(val0, zero16)
    _zero_rows(val1, zero16)
    _fire(val0, idx0, a_sp, sem0)
    _fire(val1, idx1, a_sp, sem1)

    def p2(i, _):
        _drain(val0, idx0, a_sp, sem0)
        fill(idx0, val0, 2 * i, True)
        _fire(val0, idx0, a_sp, sem0)
        _drain(val1, idx1, a_sp, sem1)
        fill(idx1, val1, 2 * i + 1, True)
        _fire(val1, idx1, a_sp, sem1)
        return 0
    lax.fori_loop(0, NB // 2, p2, 0)
    _drain(val0, idx0, a_sp, sem0)
    _drain(val1, idx1, a_sp, sem1)

    for half, (idx_b, val_b, sm) in enumerate(
            ((idx0, val0, sem0), (idx1, val1, sem1))):
        for jj in range(20):
            j = s * 40 + half * 20 + jj
            jc = jnp.minimum(j, N // L - 1)
            ok = (j * L + iota) < N
            i16 = jc * L + iota
            g16 = batch_v[pl.ds(jc * L, L)]
            dv = dinv_v[pl.ds(jc * L, L)]
            m, m2 = 2 * jj, 2 * jj + 1
            idx_b[m // 8, pl.ds((m % 8) * L, L)] = g16 * N + i16
            val_b[m // 8, pl.ds((m % 8) * L, L)] = jnp.where(ok, dv * dv, 0.0)
            idx_b[m2 // 8, pl.ds((m2 % 8) * L, L)] = CNT_OFF + g16
            val_b[m2 // 8, pl.ds((m2 % 8) * L, L)] = jnp.where(ok, 1.0, 0.0)
        for r in range(5):
            pltpu.async_copy(val_b.at[r], a_sp.at[idx_b.at[r]], sm, add=True)
    for (idx_b, val_b, sm) in ((idx0, val0, sem0), (idx1, val1, sem1)):
        for r in range(5):
            pltpu.make_async_copy(val_b.at[r], a_sp.at[idx_b.at[r]], sm).wait()
    plsc.subcore_barrier()

    h1, h2 = SL2, SL - SL2
    pltpu.sync_copy(a_sp.at[pl.ds(s * SL, h1)], stage_v)
    pltpu.sync_copy(stage_v, out_hbm.at[pl.ds(s * SL, h1)])
    pltpu.sync_copy(a_sp.at[pl.ds(s * SL + h1, h2)], stage_v.at[pl.ds(0, h2)])
    pltpu.sync_copy(stage_v.at[pl.ds(0, h2)], out_hbm.at[pl.ds(s * SL + h1, h2)])


def _tc_body(ng_ref, a_ref, cnt_ref, x_ref, w1_ref, b1_ref, w2_ref, b2_ref,
             o_ref):
    p = jnp.dot(a_ref[...], x_ref[...], preferred_element_type=jnp.float32)
    cnt = cnt_ref[...]
    z = jnp.dot(p, w1_ref[...], preferred_element_type=jnp.float32)
    sums = z + cnt * b1_ref[...]
    valid = lax.broadcasted_iota(jnp.int32, (G, 1), 0) < ng_ref[0, 0]
    sums = jnp.where(valid, sums, 0.0)
    cntv = jnp.where(valid, cnt, 0.0)
    pooled = sums / jnp.maximum(cntv, 1.0)
    logits = jnp.dot(pooled, w2_ref[...],
                     preferred_element_type=jnp.float32) + b2_ref[...]
    mx = jnp.max(logits, axis=1, keepdims=True)
    lse = mx + jnp.log(jnp.sum(jnp.exp(logits - mx), axis=1, keepdims=True))
    o_ref[...] = logits - lse


def kernel(x, edge_index, batch, num_graphs, W1, b1, W2, b2):
    mesh = plsc.VectorSubcoreMesh(core_axis_name="c", subcore_axis_name="s",
                                  num_cores=1)
    sc = pl.kernel(
        _sc_body,
        out_type=jax.ShapeDtypeStruct((ASZ,), jnp.float32),
        mesh=mesh,
        compiler_params=pltpu.CompilerParams(needs_layout_passes=False),
        scratch_types=[
            pltpu.VMEM((N,), jnp.int32),
            pltpu.VMEM((NPAD,), jnp.float32),
            pltpu.VMEM((2 * EC,), jnp.int32),
            pltpu.VMEM((NSL,), jnp.float32),
            pltpu.VMEM((RB, 128), jnp.int32),
            pltpu.VMEM((RB, 128), jnp.float32),
            pltpu.VMEM((RB, 128), jnp.int32),
            pltpu.VMEM((RB, 128), jnp.float32),
            pltpu.VMEM((ZB,), jnp.float32),
            pltpu.VMEM((SL2,), jnp.float32),
            pltpu.SemaphoreType.DMA,
            pltpu.SemaphoreType.DMA,
            pltpu.VMEM_SHARED((ASZ,), jnp.float32),
            pltpu.VMEM_SHARED((NPAD,), jnp.float32),
            pltpu.VMEM_SHARED((NPAD,), jnp.float32),
        ],
    )
    a2 = sc(edge_index[0], edge_index[1], batch)
    amat = a2[:G * N].reshape(G, N)
    cntp = a2[CNT_OFF:CNT_OFF + G].reshape(G, 1)
    ng = jnp.asarray(num_graphs, jnp.int32).reshape(1, 1)
    return pl.pallas_call(
        _tc_body,
        out_shape=jax.ShapeDtypeStruct((G, C), jnp.float32),
    )(ng, amat, cntp, x, W1, b1.reshape(1, H), W2, b2.reshape(1, C))


# speedup vs baseline: 1.3868x; 1.3868x over previous
import jax
import jax.numpy as jnp
from jax import lax
from jax.experimental import pallas as pl
from jax.experimental.pallas import tpu as pltpu
from jax.experimental.pallas import tpu_sc as plsc

N = 10000
E = 320000
D = 128
H = 64
G = 64
C = 10

NS = 16
L = 16

NPAD = 10240
NSL = NPAD // NS
CNT_OFF = G * N
ASZ = G * N + 128
SL = ASZ // NS
SL2 = 20008
EC = E // NS
CB = 1024
RB = CB // 128
NB = (EC + CB - 1) // CB
ZB = 4016


def _invsqrt(v):
    i = lax.bitcast_convert_type(v, jnp.int32)
    i = jnp.int32(0x5F3759DF) - (i >> 1)
    y = lax.bitcast_convert_type(i, jnp.float32)
    for _ in range(3):
        y = y * (1.5 - 0.5 * v * y * y)
    return y


def _sc_body(src_hbm, dst_hbm, batch_hbm, out_hbm,
             batch_v, dinv_v, edge_v, degsl_v, idx0, val0, idx1, val1,
             zeros_v, stage_v, sem0, sem1, a_sp, deg_sp, dinv_sp):
    s = lax.axis_index("s")
    iota = lax.iota(jnp.int32, L)
    zero16 = jnp.zeros((L,), jnp.float32)

    pltpu.sync_copy(batch_hbm, batch_v)
    pltpu.sync_copy(src_hbm.at[pl.ds(s * EC, EC)], edge_v.at[pl.ds(0, EC)])
    pltpu.sync_copy(dst_hbm.at[pl.ds(s * EC, EC)], edge_v.at[pl.ds(EC, EC)])

    def zloop(i, _):
        zeros_v[pl.ds(i * L, L)] = zero16
        return 0
    lax.fori_loop(0, ZB // L, zloop, 0)

    base = s * SL
    for j in range(9):
        pltpu.sync_copy(zeros_v.at[pl.ds(0, 4000)],
                        a_sp.at[pl.ds(base + j * 4000, 4000)])
    pltpu.sync_copy(zeros_v.at[pl.ds(0, SL - 36000)],
                    a_sp.at[pl.ds(base + 36000, SL - 36000)])
    pltpu.sync_copy(zeros_v.at[pl.ds(0, NSL)], deg_sp.at[pl.ds(s * NSL, NSL)])
    plsc.subcore_barrier()

    def fill(idx_b, val_b, b, with_norm):
        for k in range(CB // L):
            e0 = b * CB + k * L
            e0c = jnp.minimum(e0, EC - L)
            ok = (e0 + iota) < EC
            r, col = k // 8, (k % 8) * L
            d16 = edge_v[pl.ds(EC + e0c, L)]
            if with_norm:
                s16 = edge_v[pl.ds(e0c, L)]
                dvs = plsc.load_gather(dinv_v, [s16])
                dvd = plsc.load_gather(dinv_v, [d16])
                g16 = plsc.load_gather(batch_v, [d16])
                idx_b[r, pl.ds(col, L)] = g16 * N + s16
                val_b[r, pl.ds(col, L)] = jnp.where(ok, dvs * dvd, 0.0)
            else:
                idx_b[r, pl.ds(col, L)] = d16
                val_b[r, pl.ds(col, L)] = jnp.where(ok, 1.0, 0.0)

    def p1(i, _):
        fill(idx0, val0, 2 * i, False)
        d0 = [pltpu.async_copy(val0.at[r], deg_sp.at[idx0.at[r]], sem0,
                               add=True) for r in range(RB)]
        fill(idx1, val1, 2 * i + 1, False)
        d1 = [pltpu.async_copy(val1.at[r], deg_sp.at[idx1.at[r]], sem1,
                               add=True) for r in range(RB)]
        for d in d0 + d1:
            d.wait()
        return 0
    lax.fori_loop(0, NB // 2, p1, 0)
    plsc.subcore_barrier()

    pltpu.sync_copy(deg_sp.at[pl.ds(s * NSL, NSL)], degsl_v)

    def dloop(jj, _):
        dg = degsl_v[pl.ds(jj * L, L)] + 1.0
        dinv_v[pl.ds(s * NSL + jj * L, L)] = _invsqrt(dg)
        return 0
    lax.fori_loop(0, NSL // L, dloop, 0)
    pltpu.sync_copy(dinv_v.at[pl.ds(s * NSL, NSL)],
                    dinv_sp.at[pl.ds(s * NSL, NSL)])
    plsc.subcore_barrier()
    pltpu.sync_copy(dinv_sp, dinv_v)

    def p2(i, _):
        fill(idx0, val0, 2 * i, True)
        d0 = [pltpu.async_copy(val0.at[r], a_sp.at[idx0.at[r]], sem0,
                               add=True) for r in range(RB)]
        fill(idx1, val1, 2 * i + 1, True)
        d1 = [pltpu.async_copy(val1.at[r], a_sp.at[idx1.at[r]], sem1,
                               add=True) for r in range(RB)]
        for d in d0 + d1:
            d.wait()
        return 0
    lax.fori_loop(0, NB // 2, p2, 0)

    for half, (idx_b, val_b, sm) in enumerate(
            ((idx0, val0, sem0), (idx1, val1, sem1))):
        for jj in range(20):
            j = s * 40 + half * 20 + jj
            jc = jnp.minimum(j, N // L - 1)
            ok = (j * L + iota) < N
            i16 = jc * L + iota
            g16 = batch_v[pl.ds(jc * L, L)]
            dv = dinv_v[pl.ds(jc * L, L)]
            m, m2 = 2 * jj, 2 * jj + 1
            idx_b[m // 8, pl.ds((m % 8) * L, L)] = g16 * N + i16
            val_b[m // 8, pl.ds((m % 8) * L, L)] = jnp.where(ok, dv * dv, 0.0)
            idx_b[m2 // 8, pl.ds((m2 % 8) * L, L)] = CNT_OFF + g16
            val_b[m2 // 8, pl.ds((m2 % 8) * L, L)] = jnp.where(ok, 1.0, 0.0)
        for r in range(5):
            pltpu.async_copy(val_b.at[r], a_sp.at[idx_b.at[r]], sm, add=True)
    for (idx_b, val_b, sm) in ((idx0, val0, sem0), (idx1, val1, sem1)):
        for r in range(5):
            pltpu.make_async_copy(val_b.at[r], a_sp.at[idx_b.at[r]], sm).wait()
    plsc.subcore_barrier()

    h1, h2 = SL2, SL - SL2
    pltpu.sync_copy(a_sp.at[pl.ds(s * SL, h1)], stage_v)
    pltpu.sync_copy(stage_v, out_hbm.at[pl.ds(s * SL, h1)])
    pltpu.sync_copy(a_sp.at[pl.ds(s * SL + h1, h2)], stage_v.at[pl.ds(0, h2)])
    pltpu.sync_copy(stage_v.at[pl.ds(0, h2)], out_hbm.at[pl.ds(s * SL + h1, h2)])


def _tc_body(ng_ref, a_ref, cnt_ref, x_ref, w1_ref, b1_ref, w2_ref, b2_ref,
             o_ref):
    p = jnp.dot(a_ref[...], x_ref[...], preferred_element_type=jnp.float32)
    cnt = cnt_ref[...]
    z = jnp.dot(p, w1_ref[...], preferred_element_type=jnp.float32)
    sums = z + cnt * b1_ref[...]
    valid = lax.broadcasted_iota(jnp.int32, (G, 1), 0) < ng_ref[0, 0]
    sums = jnp.where(valid, sums, 0.0)
    cntv = jnp.where(valid, cnt, 0.0)
    pooled = sums / jnp.maximum(cntv, 1.0)
    logits = jnp.dot(pooled, w2_ref[...],
                     preferred_element_type=jnp.float32) + b2_ref[...]
    mx = jnp.max(logits, axis=1, keepdims=True)
    lse = mx + jnp.log(jnp.sum(jnp.exp(logits - mx), axis=1, keepdims=True))
    o_ref[...] = logits - lse


def kernel(x, edge_index, batch, num_graphs, W1, b1, W2, b2):
    mesh = plsc.VectorSubcoreMesh(core_axis_name="c", subcore_axis_name="s",
                                  num_cores=1)
    sc = pl.kernel(
        _sc_body,
        out_type=jax.ShapeDtypeStruct((ASZ,), jnp.float32),
        mesh=mesh,
        compiler_params=pltpu.CompilerParams(needs_layout_passes=False),
        scratch_types=[
            pltpu.VMEM((N,), jnp.int32),
            pltpu.VMEM((NPAD,), jnp.float32),
            pltpu.VMEM((2 * EC,), jnp.int32),
            pltpu.VMEM((NSL,), jnp.float32),
            pltpu.VMEM((RB, 128), jnp.int32),
            pltpu.VMEM((RB, 128), jnp.float32),
            pltpu.VMEM((RB, 128), jnp.int32),
            pltpu.VMEM((RB, 128), jnp.float32),
            pltpu.VMEM((ZB,), jnp.float32),
            pltpu.VMEM((SL2,), jnp.float32),
            pltpu.SemaphoreType.DMA,
            pltpu.SemaphoreType.DMA,
            pltpu.VMEM_SHARED((ASZ,), jnp.float32),
            pltpu.VMEM_SHARED((NPAD,), jnp.float32),
            pltpu.VMEM_SHARED((NPAD,), jnp.float32),
        ],
    )
    a2 = sc(edge_index[0], edge_index[1], batch)
    amat = a2[:G * N].reshape(G, N)
    cntp = a2[CNT_OFF:CNT_OFF + G].reshape(G, 1)
    ng = jnp.asarray(num_graphs, jnp.int32).reshape(1, 1)
    return pl.pallas_call(
        _tc_body,
        out_shape=jax.ShapeDtypeStruct((G, C), jnp.float32),
    )(ng, amat, cntp, x, W1, b1.reshape(1, H), W2, b2.reshape(1, C))


# speedup vs baseline: 1.3891x; 1.0016x over previous
import jax
import jax.numpy as jnp
from jax import lax
from jax.experimental import pallas as pl
from jax.experimental.pallas import tpu as pltpu
from jax.experimental.pallas import tpu_sc as plsc

N = 10000
E = 320000
D = 128
H = 64
G = 64
C = 10

NS = 16
L = 16

NPAD = 10240
NSL = NPAD // NS
CNT_OFF = G * N
ASZ = G * N + 128
SL = ASZ // NS
SL2 = 20008
EC = E // NS
CB = 1024
RB = CB // 128
NB = (EC + CB - 1) // CB
ZB = 4016


def _invsqrt(v):
    i = lax.bitcast_convert_type(v, jnp.int32)
    i = jnp.int32(0x5F3759DF) - (i >> 1)
    y = lax.bitcast_convert_type(i, jnp.float32)
    for _ in range(3):
        y = y * (1.5 - 0.5 * v * y * y)
    return y


def _sc_body(src_hbm, dst_hbm, batch_hbm, out_hbm,
             batch_v, dinv_v, edge_v, degsl_v, idx0, val0, idx1, val1,
             zeros_v, stage_v, sem0, sem1, a_sp, deg_sp, dinv_sp):
    s = lax.axis_index("s")
    iota = lax.iota(jnp.int32, L)
    zero16 = jnp.zeros((L,), jnp.float32)

    pltpu.sync_copy(batch_hbm, batch_v)
    pltpu.sync_copy(src_hbm.at[pl.ds(s * EC, EC)], edge_v.at[pl.ds(0, EC)])
    pltpu.sync_copy(dst_hbm.at[pl.ds(s * EC, EC)], edge_v.at[pl.ds(EC, EC)])

    def zloop(i, _):
        zeros_v[pl.ds(i * L, L)] = zero16
        return 0
    lax.fori_loop(0, ZB // L, zloop, 0)

    base = s * SL
    for j in range(9):
        pltpu.sync_copy(zeros_v.at[pl.ds(0, 4000)],
                        a_sp.at[pl.ds(base + j * 4000, 4000)])
    pltpu.sync_copy(zeros_v.at[pl.ds(0, SL - 36000)],
                    a_sp.at[pl.ds(base + 36000, SL - 36000)])
    pltpu.sync_copy(zeros_v.at[pl.ds(0, NSL)], deg_sp.at[pl.ds(s * NSL, NSL)])
    plsc.subcore_barrier()

    def fill(idx_b, val_b, b, with_norm):
        for k in range(CB // L):
            e0 = b * CB + k * L
            e0c = jnp.minimum(e0, EC - L)
            ok = (e0 + iota) < EC
            col = k * L
            d16 = edge_v[pl.ds(EC + e0c, L)]
            if with_norm:
                s16 = edge_v[pl.ds(e0c, L)]
                dvs = plsc.load_gather(dinv_v, [s16])
                dvd = plsc.load_gather(dinv_v, [d16])
                g16 = plsc.load_gather(batch_v, [d16])
                idx_b[pl.ds(col, L)] = g16 * N + s16
                val_b[pl.ds(col, L)] = jnp.where(ok, dvs * dvd, 0.0)
            else:
                idx_b[pl.ds(col, L)] = d16
                val_b[pl.ds(col, L)] = jnp.where(ok, 1.0, 0.0)

    def p1(i, _):
        fill(idx0, val0, 2 * i, False)
        d0 = pltpu.async_copy(val0, deg_sp.at[idx0], sem0, add=True)
        fill(idx1, val1, 2 * i + 1, False)
        d1 = pltpu.async_copy(val1, deg_sp.at[idx1], sem1, add=True)
        d0.wait()
        d1.wait()
        return 0
    lax.fori_loop(0, NB // 2, p1, 0)
    plsc.subcore_barrier()

    pltpu.sync_copy(deg_sp.at[pl.ds(s * NSL, NSL)], degsl_v)

    def dloop(jj, _):
        dg = degsl_v[pl.ds(jj * L, L)] + 1.0
        dinv_v[pl.ds(s * NSL + jj * L, L)] = _invsqrt(dg)
        return 0
    lax.fori_loop(0, NSL // L, dloop, 0)
    pltpu.sync_copy(dinv_v.at[pl.ds(s * NSL, NSL)],
                    dinv_sp.at[pl.ds(s * NSL, NSL)])
    plsc.subcore_barrier()
    pltpu.sync_copy(dinv_sp, dinv_v)

    def p2(i, _):
        fill(idx0, val0, 2 * i, True)
        d0 = pltpu.async_copy(val0, a_sp.at[idx0], sem0, add=True)
        fill(idx1, val1, 2 * i + 1, True)
        d1 = pltpu.async_copy(val1, a_sp.at[idx1], sem1, add=True)
        d0.wait()
        d1.wait()
        return 0
    lax.fori_loop(0, NB // 2, p2, 0)

    descs = []
    for half, (idx_b, val_b, sm) in enumerate(
            ((idx0, val0, sem0), (idx1, val1, sem1))):
        for jj in range(20):
            j = s * 40 + half * 20 + jj
            jc = jnp.minimum(j, N // L - 1)
            ok = (j * L + iota) < N
            i16 = jc * L + iota
            g16 = batch_v[pl.ds(jc * L, L)]
            dv = dinv_v[pl.ds(jc * L, L)]
            m, m2 = 2 * jj, 2 * jj + 1
            idx_b[pl.ds(m * L, L)] = g16 * N + i16
            val_b[pl.ds(m * L, L)] = jnp.where(ok, dv * dv, 0.0)
            idx_b[pl.ds(m2 * L, L)] = CNT_OFF + g16
            val_b[pl.ds(m2 * L, L)] = jnp.where(ok, 1.0, 0.0)
        for m in range(40, CB // L):
            val_b[pl.ds(m * L, L)] = zero16
        descs.append(pltpu.async_copy(val_b, a_sp.at[idx_b], sm, add=True))
    for d in descs:
        d.wait()
    plsc.subcore_barrier()

    h1, h2 = SL2, SL - SL2
    pltpu.sync_copy(a_sp.at[pl.ds(s * SL, h1)], stage_v)
    pltpu.sync_copy(stage_v, out_hbm.at[pl.ds(s * SL, h1)])
    pltpu.sync_copy(a_sp.at[pl.ds(s * SL + h1, h2)], stage_v.at[pl.ds(0, h2)])
    pltpu.sync_copy(stage_v.at[pl.ds(0, h2)], out_hbm.at[pl.ds(s * SL + h1, h2)])


def _tc_body(ng_ref, a_ref, cnt_ref, x_ref, w1_ref, b1_ref, w2_ref, b2_ref,
             o_ref):
    p = jnp.dot(a_ref[...], x_ref[...], preferred_element_type=jnp.float32)
    cnt = cnt_ref[...]
    z = jnp.dot(p, w1_ref[...], preferred_element_type=jnp.float32)
    sums = z + cnt * b1_ref[...]
    valid = lax.broadcasted_iota(jnp.int32, (G, 1), 0) < ng_ref[0, 0]
    sums = jnp.where(valid, sums, 0.0)
    cntv = jnp.where(valid, cnt, 0.0)
    pooled = sums / jnp.maximum(cntv, 1.0)
    logits = jnp.dot(pooled, w2_ref[...],
                     preferred_element_type=jnp.float32) + b2_ref[...]
    mx = jnp.max(logits, axis=1, keepdims=True)
    lse = mx + jnp.log(jnp.sum(jnp.exp(logits - mx), axis=1, keepdims=True))
    o_ref[...] = logits - lse


def kernel(x, edge_index, batch, num_graphs, W1, b1, W2, b2):
    mesh = plsc.VectorSubcoreMesh(core_axis_name="c", subcore_axis_name="s",
                                  num_cores=1)
    sc = pl.kernel(
        _sc_body,
        out_type=jax.ShapeDtypeStruct((ASZ,), jnp.float32),
        mesh=mesh,
        compiler_params=pltpu.CompilerParams(needs_layout_passes=False),
        scratch_types=[
            pltpu.VMEM((N,), jnp.int32),
            pltpu.VMEM((NPAD,), jnp.float32),
            pltpu.VMEM((2 * EC,), jnp.int32),
            pltpu.VMEM((NSL,), jnp.float32),
            pltpu.VMEM((CB,), jnp.int32),
            pltpu.VMEM((CB,), jnp.float32),
            pltpu.VMEM((CB,), jnp.int32),
            pltpu.VMEM((CB,), jnp.float32),
            pltpu.VMEM((ZB,), jnp.float32),
            pltpu.VMEM((SL2,), jnp.float32),
            pltpu.SemaphoreType.DMA,
            pltpu.SemaphoreType.DMA,
            pltpu.VMEM_SHARED((ASZ,), jnp.float32),
            pltpu.VMEM_SHARED((NPAD,), jnp.float32),
            pltpu.VMEM_SHARED((NPAD,), jnp.float32),
        ],
    )
    a2 = sc(edge_index[0], edge_index[1], batch)
    amat = a2[:G * N].reshape(G, N)
    cntp = a2[CNT_OFF:CNT_OFF + G].reshape(G, 1)
    ng = jnp.asarray(num_graphs, jnp.int32).reshape(1, 1)
    return pl.pallas_call(
        _tc_body,
        out_shape=jax.ShapeDtypeStruct((G, C), jnp.float32),
    )(ng, amat, cntp, x, W1, b1.reshape(1, H), W2, b2.reshape(1, C))


# speedup vs baseline: 1.3926x; 1.0025x over previous
import jax
import jax.numpy as jnp
from jax import lax
from jax.experimental import pallas as pl
from jax.experimental.pallas import tpu as pltpu
from jax.experimental.pallas import tpu_sc as plsc

N = 10000
E = 320000
D = 128
H = 64
G = 64
C = 10

NS = 16
L = 16

NPAD = 10240
NSL = NPAD // NS
CNT_OFF = G * N
ASZ = G * N + 128
SL = ASZ // NS
SL2 = 8008
EC = E // NS
CB = 1024
RB = CB // 128
NB = (EC + CB - 1) // CB
ZB = 4016


def _invsqrt(v):
    i = lax.bitcast_convert_type(v, jnp.int32)
    i = jnp.int32(0x5F3759DF) - (i >> 1)
    y = lax.bitcast_convert_type(i, jnp.float32)
    for _ in range(3):
        y = y * (1.5 - 0.5 * v * y * y)
    return y


def _sc_body(src_hbm, dst_hbm, batch_hbm, out_hbm,
             batch_v, dinv_v, edge_v, degsl_v, idx0, val0, idx1, val1,
             idx2, val2, idx3, val3, onesf_v, onest_v,
             zeros_v, stage_v, sem0, sem1, sem2, sem3,
             a_sp, deg_sp, dinv_sp):
    s = lax.axis_index("s")
    iota = lax.iota(jnp.int32, L)
    zero16 = jnp.zeros((L,), jnp.float32)
    idxs = (idx0, idx1, idx2, idx3)
    vals = (val0, val1, val2, val3)
    sems = (sem0, sem1, sem2, sem3)

    pltpu.sync_copy(batch_hbm, batch_v)
    pltpu.sync_copy(src_hbm.at[pl.ds(s * EC, EC)], edge_v.at[pl.ds(0, EC)])
    pltpu.sync_copy(dst_hbm.at[pl.ds(s * EC, EC)], edge_v.at[pl.ds(EC, EC)])

    def zloop(i, _):
        zeros_v[pl.ds(i * L, L)] = zero16
        return 0
    lax.fori_loop(0, ZB // L, zloop, 0)

    tail_valid = EC - (NB - 1) * CB

    def oloop(i, _):
        onesf_v[pl.ds(i * L, L)] = jnp.full((L,), 1.0, jnp.float32)
        onest_v[pl.ds(i * L, L)] = jnp.where(i * L + iota < tail_valid,
                                             1.0, 0.0)
        return 0
    lax.fori_loop(0, CB // L, oloop, 0)

    base = s * SL
    for j in range(9):
        pltpu.sync_copy(zeros_v.at[pl.ds(0, 4000)],
                        a_sp.at[pl.ds(base + j * 4000, 4000)])
    pltpu.sync_copy(zeros_v.at[pl.ds(0, SL - 36000)],
                    a_sp.at[pl.ds(base + 36000, SL - 36000)])
    pltpu.sync_copy(zeros_v.at[pl.ds(0, NSL)], deg_sp.at[pl.ds(s * NSL, NSL)])
    plsc.subcore_barrier()

    def fill_deg(idx_b, b):
        for k in range(CB // L):
            e0c = jnp.minimum(b * CB + k * L, EC - L)
            idx_b[pl.ds(k * L, L)] = edge_v[pl.ds(EC + e0c, L)]

    def fill(idx_b, val_b, b):
        for k in range(CB // L):
            e0 = b * CB + k * L
            e0c = jnp.minimum(e0, EC - L)
            ok = (e0 + iota) < EC
            col = k * L
            d16 = edge_v[pl.ds(EC + e0c, L)]
            s16 = edge_v[pl.ds(e0c, L)]
            dvs = plsc.load_gather(dinv_v, [s16])
            dvd = plsc.load_gather(dinv_v, [d16])
            g16 = plsc.load_gather(batch_v, [d16])
            idx_b[pl.ds(col, L)] = g16 * N + s16
            val_b[pl.ds(col, L)] = jnp.where(ok, dvs * dvd, 0.0)

    def p1(i, _):
        descs = []
        for q in range(4):
            fill_deg(idxs[q], 4 * i + q)
            descs.append(pltpu.async_copy(onesf_v, deg_sp.at[idxs[q]],
                                          sems[q], add=True))
        for d in descs:
            d.wait()
        return 0
    lax.fori_loop(0, NB // 4 - 1, p1, 0)
    descs = []
    for q in range(4):
        fill_deg(idxs[q], NB - 4 + q)
        vref = onest_v if q == 3 else onesf_v
        descs.append(pltpu.async_copy(vref, deg_sp.at[idxs[q]],
                                      sems[q], add=True))
    for d in descs:
        d.wait()
    plsc.subcore_barrier()

    pltpu.sync_copy(deg_sp.at[pl.ds(s * NSL, NSL)], degsl_v)

    def dloop(jj, _):
        dg = degsl_v[pl.ds(jj * L, L)] + 1.0
        dinv_v[pl.ds(s * NSL + jj * L, L)] = _invsqrt(dg)
        return 0
    lax.fori_loop(0, NSL // L, dloop, 0)
    pltpu.sync_copy(dinv_v.at[pl.ds(s * NSL, NSL)],
                    dinv_sp.at[pl.ds(s * NSL, NSL)])
    plsc.subcore_barrier()
    pltpu.sync_copy(dinv_sp, dinv_v)

    def p2(i, _):
        descs = []
        for q in range(4):
            fill(idxs[q], vals[q], 4 * i + q)
            descs.append(pltpu.async_copy(vals[q], a_sp.at[idxs[q]],
                                          sems[q], add=True))
        for d in descs:
            d.wait()
        return 0
    lax.fori_loop(0, NB // 4, p2, 0)

    descs = []
    for half, (idx_b, val_b, sm) in enumerate(
            ((idx0, val0, sem0), (idx1, val1, sem1))):
        for jj in range(20):
            j = s * 40 + half * 20 + jj
            jc = jnp.minimum(j, N // L - 1)
            ok = (j * L + iota) < N
            i16 = jc * L + iota
            g16 = batch_v[pl.ds(jc * L, L)]
            dv = dinv_v[pl.ds(jc * L, L)]
            m, m2 = 2 * jj, 2 * jj + 1
            idx_b[pl.ds(m * L, L)] = g16 * N + i16
            val_b[pl.ds(m * L, L)] = jnp.where(ok, dv * dv, 0.0)
            idx_b[pl.ds(m2 * L, L)] = CNT_OFF + g16
            val_b[pl.ds(m2 * L, L)] = jnp.where(ok, 1.0, 0.0)
        for m in range(40, CB // L):
            val_b[pl.ds(m * L, L)] = zero16
        descs.append(pltpu.async_copy(val_b, a_sp.at[idx_b], sm, add=True))
    for d in descs:
        d.wait()
    plsc.subcore_barrier()

    pltpu.sync_copy(a_sp.at[pl.ds(s * SL, SL2)], stage_v)
    pltpu.sync_copy(stage_v, out_hbm.at[pl.ds(s * SL, SL2)])
    for j in range(4):
        off = s * SL + SL2 + j * 8000
        pltpu.sync_copy(a_sp.at[pl.ds(off, 8000)], stage_v.at[pl.ds(0, 8000)])
        pltpu.sync_copy(stage_v.at[pl.ds(0, 8000)], out_hbm.at[pl.ds(off, 8000)])


def _tc_body(ng_ref, a_ref, cnt_ref, x_ref, w1_ref, b1_ref, w2_ref, b2_ref,
             o_ref):
    p = jnp.dot(a_ref[...], x_ref[...], preferred_element_type=jnp.float32)
    cnt = cnt_ref[...]
    z = jnp.dot(p, w1_ref[...], preferred_element_type=jnp.float32)
    sums = z + cnt * b1_ref[...]
    valid = lax.broadcasted_iota(jnp.int32, (G, 1), 0) < ng_ref[0, 0]
    sums = jnp.where(valid, sums, 0.0)
    cntv = jnp.where(valid, cnt, 0.0)
    pooled = sums / jnp.maximum(cntv, 1.0)
    logits = jnp.dot(pooled, w2_ref[...],
                     preferred_element_type=jnp.float32) + b2_ref[...]
    mx = jnp.max(logits, axis=1, keepdims=True)
    lse = mx + jnp.log(jnp.sum(jnp.exp(logits - mx), axis=1, keepdims=True))
    o_ref[...] = logits - lse


def kernel(x, edge_index, batch, num_graphs, W1, b1, W2, b2):
    mesh = plsc.VectorSubcoreMesh(core_axis_name="c", subcore_axis_name="s",
                                  num_cores=1)
    sc = pl.kernel(
        _sc_body,
        out_type=jax.ShapeDtypeStruct((ASZ,), jnp.float32),
        mesh=mesh,
        compiler_params=pltpu.CompilerParams(needs_layout_passes=False),
        scratch_types=[
            pltpu.VMEM((N,), jnp.int32),
            pltpu.VMEM((NPAD,), jnp.float32),
            pltpu.VMEM((2 * EC,), jnp.int32),
            pltpu.VMEM((NSL,), jnp.float32),
            pltpu.VMEM((CB,), jnp.int32),
            pltpu.VMEM((CB,), jnp.float32),
            pltpu.VMEM((CB,), jnp.int32),
            pltpu.VMEM((CB,), jnp.float32),
            pltpu.VMEM((CB,), jnp.int32),
            pltpu.VMEM((CB,), jnp.float32),
            pltpu.VMEM((CB,), jnp.int32),
            pltpu.VMEM((CB,), jnp.float32),
            pltpu.VMEM((CB,), jnp.float32),
            pltpu.VMEM((CB,), jnp.float32),
            pltpu.VMEM((ZB,), jnp.float32),
            pltpu.VMEM((SL2,), jnp.float32),
            pltpu.SemaphoreType.DMA,
            pltpu.SemaphoreType.DMA,
            pltpu.SemaphoreType.DMA,
            pltpu.SemaphoreType.DMA,
            pltpu.VMEM_SHARED((ASZ,), jnp.float32),
            pltpu.VMEM_SHARED((NPAD,), jnp.float32),
            pltpu.VMEM_SHARED((NPAD,), jnp.float32),
        ],
    )
    a2 = sc(edge_index[0], edge_index[1], batch)
    amat = a2[:G * N].reshape(G, N)
    cntp = a2[CNT_OFF:CNT_OFF + G].reshape(G, 1)
    ng = jnp.asarray(num_graphs, jnp.int32).reshape(1, 1)
    return pl.pallas_call(
        _tc_body,
        out_shape=jax.ShapeDtypeStruct((G, C), jnp.float32),
    )(ng, amat, cntp, x, W1, b1.reshape(1, H), W2, b2.reshape(1, C))
